# Initial kernel scaffold; baseline (speedup 1.0000x reference)
import jax
import jax.numpy as jnp
from jax.experimental import pallas as pl


def kernel(x, edge_index, batch, pos, W_ae, b_ae, Wd1, bd1, Wd2, bd2, Wm1, bm1, Wm2, bm2, Wc, bc, Wl, bl, Wf1, bf1, Wf2, bf2):
    raise NotImplementedError("write your pallas kernel here")


# speedup vs baseline: 2.3350x; 2.3350x over previous
import functools

import jax
import jax.numpy as jnp
from jax import lax
from jax.experimental import pallas as pl
from jax.experimental.pallas import tpu as pltpu
from jax.experimental.pallas import tpu_sc as plsc

_NC, _NS = 2, 16
_NW = _NC * _NS
_CH = 128
_G = 256
_NT = 1000
_UT = 80
_ET = 512


def _mesh():
    return plsc.VectorSubcoreMesh(
        core_axis_name="c", subcore_axis_name="s", num_cores=_NC, num_subcores=_NS
    )


def _make_dist_kernel(n, e_pad):
    per_tile = e_pad // _NW
    n_chunks = per_tile // _CH

    @functools.partial(
        pl.kernel,
        out_type=jax.ShapeDtypeStruct((e_pad,), jnp.float32),
        mesh=_mesh(),
        scratch_types=[
            pltpu.VMEM((n,), jnp.float32),
            pltpu.VMEM((n,), jnp.float32),
            pltpu.VMEM((n,), jnp.float32),
            pltpu.VMEM((_CH,), jnp.int32),
            pltpu.VMEM((_CH,), jnp.int32),
            pltpu.VMEM((_CH,), jnp.float32),
        ],
        compiler_params=pltpu.CompilerParams(needs_layout_passes=False),
    )
    def dist_kernel(px_hbm, py_hbm, pz_hbm, row_hbm, col_hbm, sq_hbm,
                    px, py, pz, ir, ic, ob):
        cid = lax.axis_index("c")
        sid = lax.axis_index("s")
        wid = cid * _NS + sid
        pltpu.sync_copy(px_hbm, px)
        pltpu.sync_copy(py_hbm, py)
        pltpu.sync_copy(pz_hbm, pz)
        base0 = wid * per_tile

        def chunk(t, carry):
            base = base0 + t * _CH
            pltpu.sync_copy(row_hbm.at[pl.ds(base, _CH)], ir)
            pltpu.sync_copy(col_hbm.at[pl.ds(base, _CH)], ic)
            for g in range(_CH // 16):
                sl = pl.ds(g * 16, 16)
                rv = ir[sl]
                cv = ic[sl]
                dx = plsc.load_gather(px, [rv]) - plsc.load_gather(px, [cv])
                dy = plsc.load_gather(py, [rv]) - plsc.load_gather(py, [cv])
                dz = plsc.load_gather(pz, [rv]) - plsc.load_gather(pz, [cv])
                ob[sl] = dx * dx + dy * dy + dz * dz
            pltpu.sync_copy(ob, sq_hbm.at[pl.ds(base, _CH)])
            return carry

        lax.fori_loop(0, n_chunks, chunk, 0)

    return dist_kernel


def _make_edge_agg_kernel(n_pad, e_pad, h):
    per_tile = e_pad // _NW
    n_chunks = per_tile // _CH
    rows_per_sub = n_pad // _NS
    zr = 128
    assert rows_per_sub % zr == 0

    @functools.partial(
        pl.kernel,
        out_type=jax.ShapeDtypeStruct((_NC * n_pad, h), jnp.float32),
        mesh=_mesh(),
        scratch_types=[
            pltpu.VMEM((_CH,), jnp.int32),
            pltpu.VMEM((_CH,), jnp.int32),
            pltpu.VMEM((_CH, h), jnp.float32),
            pltpu.VMEM((_CH, h), jnp.float32),
            pltpu.VMEM_SHARED((n_pad, h), jnp.float32),
            pltpu.SemaphoreType.DMA,
        ],
        compiler_params=pltpu.CompilerParams(needs_layout_passes=False),
    )
    def edge_kernel(m_hbm, row_hbm, col_hbm, hc_hbm, agg_hbm,
                    ir, ic, hg, mb, aggsh, sem):
        cid = lax.axis_index("c")
        sid = lax.axis_index("s")
        wid = cid * _NS + sid

        def zrow(i, carry):
            for j in range(h // 16):
                mb[i, pl.ds(j * 16, 16)] = jnp.zeros((16,), jnp.float32)
            return carry

        lax.fori_loop(0, zr, zrow, 0)
        for k in range(rows_per_sub // zr):
            pltpu.sync_copy(mb, aggsh.at[pl.ds(sid * rows_per_sub + k * zr, zr)])
        plsc.subcore_barrier()

        base0 = wid * per_tile

        def chunk(t, carry):
            base = base0 + t * _CH
            pltpu.sync_copy(row_hbm.at[pl.ds(base, _CH)], ir)
            pltpu.sync_copy(col_hbm.at[pl.ds(base, _CH)], ic)
            pltpu.async_copy(hc_hbm.at[ir], hg, sem).wait()
            pltpu.sync_copy(m_hbm.at[pl.ds(base, _CH)], mb)

            def mulrow(i, c2):
                for j in range(h // 16):
                    sl = pl.ds(j * 16, 16)
                    mb[i, sl] = mb[i, sl] * hg[i, sl]
                return c2

            lax.fori_loop(0, _CH, mulrow, 0)
            pltpu.sync_copy(mb, aggsh.at[ic], add=True)
            return carry

        lax.fori_loop(0, n_chunks, chunk, 0)
        plsc.subcore_barrier()
        pltpu.sync_copy(
            aggsh.at[pl.ds(sid * rows_per_sub, rows_per_sub)],
            agg_hbm.at[pl.ds(cid * n_pad + sid * rows_per_sub, rows_per_sub)],
        )

    return edge_kernel


def _embed_body(x_ref, wae_ref, bae_ref, wc_ref, bc_ref, h_ref, hc_ref):
    h = jnp.dot(x_ref[...], wae_ref[...], preferred_element_type=jnp.float32)
    h = h + bae_ref[...][None, :]
    h_ref[...] = h
    hc_ref[...] = (
        jnp.dot(h, wc_ref[...], preferred_element_type=jnp.float32)
        + bc_ref[...][None, :]
    )


def _embed_call(x, wae, bae, wc0, bc0, n, f, h):
    return pl.pallas_call(
        _embed_body,
        grid=(n // _NT,),
        in_specs=[
            pl.BlockSpec((_NT, f), lambda i: (i, 0)),
            pl.BlockSpec((f, h), lambda i: (0, 0)),
            pl.BlockSpec((h,), lambda i: (0,)),
            pl.BlockSpec((h, h), lambda i: (0, 0)),
            pl.BlockSpec((h,), lambda i: (0,)),
        ],
        out_specs=[
            pl.BlockSpec((_NT, h), lambda i: (i, 0)),
            pl.BlockSpec((_NT, h), lambda i: (i, 0)),
        ],
        out_shape=[
            jax.ShapeDtypeStruct((n, h), jnp.float32),
            jax.ShapeDtypeStruct((n, h), jnp.float32),
        ],
    )(x, wae, bae, wc0, bc0)


def _msg_body(n_valid, sq_ref, wd1_ref, bd1_ref, wd2_ref, wm1_ref, bm1_ref,
              bd2_ref, wm2_ref, bm2_ref, m_ref, a_scr, c_scr):
    i = pl.program_id(0)

    @pl.when(i == 0)
    def _():
        a_scr[...] = jnp.dot(
            wd2_ref[...], wm1_ref[...], preferred_element_type=jnp.float32
        )
        c_scr[...] = (
            jnp.dot(
                bd2_ref[...][None, :], wm1_ref[...],
                preferred_element_type=jnp.float32,
            )
            + bm1_ref[...][None, :]
        )

    dist = jnp.sqrt(sq_ref[...])
    t = jax.nn.softplus(dist * wd1_ref[...] + bd1_ref[...][None, :])
    u = jax.nn.softplus(
        jnp.dot(t, a_scr[...], preferred_element_type=jnp.float32) + c_scr[...]
    )
    m = (
        jnp.dot(u, wm2_ref[...], preferred_element_type=jnp.float32)
        + bm2_ref[...][None, :]
    )
    m_ref[...] = jnp.where(i < n_valid, m, jnp.zeros_like(m))


def _msg_call(sq2, wd1, bd1, wd2, wm1_l, bm1_l, bd2, wm2_l, bm2_l, e, e_pad, h):
    n_valid = e // _ET
    assert e % _ET == 0 and e_pad % _ET == 0
    return pl.pallas_call(
        functools.partial(_msg_body, n_valid),
        grid=(e_pad // _ET,),
        in_specs=[
            pl.BlockSpec((_ET, 1), lambda i: (i, 0)),
            pl.BlockSpec((1, h), lambda i: (0, 0)),
            pl.BlockSpec((h,), lambda i: (0,)),
            pl.BlockSpec((h, h), lambda i: (0, 0)),
            pl.BlockSpec((h, h), lambda i: (0, 0)),
            pl.BlockSpec((h,), lambda i: (0,)),
            pl.BlockSpec((h,), lambda i: (0,)),
            pl.BlockSpec((h, h), lambda i: (0, 0)),
            pl.BlockSpec((h,), lambda i: (0,)),
        ],
        out_specs=pl.BlockSpec((_ET, h), lambda i: (i, 0)),
        out_shape=jax.ShapeDtypeStruct((e_pad, h), jnp.float32),
        scratch_shapes=[
            pltpu.VMEM((h, h), jnp.float32),
            pltpu.VMEM((1, h), jnp.float32),
        ],
    )(sq2, wd1, bd1, wd2, wm1_l, bm1_l, bd2, wm2_l, bm2_l)


def _upd_body(h_ref, a0_ref, a1_ref, wl_ref, bl_ref, wc_ref, bc_ref,
              hn_ref, hc_ref):
    agg = a0_ref[...] + a1_ref[...]
    hn = jax.nn.softplus(
        h_ref[...]
        + jnp.dot(agg, wl_ref[...], preferred_element_type=jnp.float32)
        + bl_ref[...][None, :]
    )
    hn_ref[...] = hn
    hc_ref[...] = (
        jnp.dot(hn, wc_ref[...], preferred_element_type=jnp.float32)
        + bc_ref[...][None, :]
    )


def _upd_call(hcur, aggp, wl_l, bl_l, wc_next, bc_next, n, n_pad, h):
    nb = n // _UT
    off = n_pad // _UT
    return pl.pallas_call(
        _upd_body,
        grid=(nb,),
        in_specs=[
            pl.BlockSpec((_UT, h), lambda i: (i, 0)),
            pl.BlockSpec((_UT, h), lambda i: (i, 0)),
            pl.BlockSpec((_UT, h), lambda i, off=off: (i + off, 0)),
            pl.BlockSpec((h, h), lambda i: (0, 0)),
            pl.BlockSpec((h,), lambda i: (0,)),
            pl.BlockSpec((h, h), lambda i: (0, 0)),
            pl.BlockSpec((h,), lambda i: (0,)),
        ],
        out_specs=[
            pl.BlockSpec((_UT, h), lambda i: (i, 0)),
            pl.BlockSpec((_UT, h), lambda i: (i, 0)),
        ],
        out_shape=[
            jax.ShapeDtypeStruct((n, h), jnp.float32),
            jax.ShapeDtypeStruct((n, h), jnp.float32),
        ],
    )(hcur, aggp, aggp, wl_l, bl_l, wc_next, bc_next)


def _upd_last_body(h_ref, a0_ref, a1_ref, wl_ref, bl_ref, hn_ref):
    agg = a0_ref[...] + a1_ref[...]
    hn_ref[...] = jax.nn.softplus(
        h_ref[...]
        + jnp.dot(agg, wl_ref[...], preferred_element_type=jnp.float32)
        + bl_ref[...][None, :]
    )


def _upd_last_call(hcur, aggp, wl_l, bl_l, n, n_pad, h):
    nb = n // _UT
    off = n_pad // _UT
    return pl.pallas_call(
        _upd_last_body,
        grid=(nb,),
        in_specs=[
            pl.BlockSpec((_UT, h), lambda i: (i, 0)),
            pl.BlockSpec((_UT, h), lambda i: (i, 0)),
            pl.BlockSpec((_UT, h), lambda i, off=off: (i + off, 0)),
            pl.BlockSpec((h, h), lambda i: (0, 0)),
            pl.BlockSpec((h,), lambda i: (0,)),
        ],
        out_specs=pl.BlockSpec((_UT, h), lambda i: (i, 0)),
        out_shape=jax.ShapeDtypeStruct((n, h), jnp.float32),
    )(hcur, aggp, aggp, wl_l, bl_l)


def _pool_body(h_ref, b_ref, wf1_ref, bf1_ref, wf2_ref, bf2_ref, out_ref, acc):
    i = pl.program_id(0)
    nb = pl.num_programs(0)
    onehot = (
        b_ref[...] == lax.broadcasted_iota(jnp.int32, (1, _G), 1)
    ).astype(jnp.float32)
    part = lax.dot_general(
        onehot, h_ref[...], (((0,), (0,)), ((), ())),
        preferred_element_type=jnp.float32,
    )
    prev = jnp.where(i == 0, jnp.zeros_like(part), acc[...])
    acc[...] = prev + part

    @pl.when(i == nb - 1)
    def _():
        hid = jax.nn.softplus(
            jnp.dot(acc[...], wf1_ref[...], preferred_element_type=jnp.float32)
            + bf1_ref[...][None, :]
        )
        out_ref[...] = (
            jnp.dot(hid, wf2_ref[...], preferred_element_type=jnp.float32)
            + bf2_ref[...][None, :]
        )


def _pool_call(hcur, batch2, wf1, bf1, wf2, bf2, n, h):
    h2 = h // 2
    return pl.pallas_call(
        _pool_body,
        grid=(n // _NT,),
        in_specs=[
            pl.BlockSpec((_NT, h), lambda i: (i, 0)),
            pl.BlockSpec((_NT, 1), lambda i: (i, 0)),
            pl.BlockSpec((h, h2), lambda i: (0, 0)),
            pl.BlockSpec((h2,), lambda i: (0,)),
            pl.BlockSpec((h2, 1), lambda i: (0, 0)),
            pl.BlockSpec((1,), lambda i: (0,)),
        ],
        out_specs=pl.BlockSpec((_G, 1), lambda i: (0, 0)),
        out_shape=jax.ShapeDtypeStruct((_G, 1), jnp.float32),
        scratch_shapes=[pltpu.VMEM((_G, h), jnp.float32)],
    )(hcur, batch2, wf1, bf1, wf2, bf2)


def kernel(x, edge_index, batch, pos, W_ae, b_ae, Wd1, bd1, Wd2, bd2,
           Wm1, bm1, Wm2, bm2, Wc, bc, Wl, bl, Wf1, bf1, Wf2, bf2):
    n, f = x.shape
    h = W_ae.shape[1]
    num_layers = Wm1.shape[0]
    e = edge_index.shape[1]

    quantum = _NW * _CH
    e_pad = ((e + quantum - 1) // quantum) * quantum
    pad = e_pad - e
    row = edge_index[0]
    col = edge_index[1]
    rowp = jnp.concatenate([row, jnp.zeros((pad,), jnp.int32)])
    colp = jnp.concatenate([col, jnp.zeros((pad,), jnp.int32)])
    px = jnp.asarray(pos[:, 0])
    py = jnp.asarray(pos[:, 1])
    pz = jnp.asarray(pos[:, 2])
    batch2 = batch[:, None]

    n_quantum = _NS * 128
    n_pad = ((n + n_quantum - 1) // n_quantum) * n_quantum

    sq = _make_dist_kernel(n, e_pad)(px, py, pz, rowp, colp)
    sq2 = sq[:, None]

    edge_agg = _make_edge_agg_kernel(n_pad, e_pad, h)

    hcur, hc = _embed_call(x, W_ae, b_ae, Wc[0], bc[0], n, f, h)
    for l in range(num_layers):
        m = _msg_call(sq2, Wd1, bd1, Wd2, Wm1[l], bm1[l], bd2, Wm2[l], bm2[l],
                      e, e_pad, h)
        aggp = edge_agg(m, rowp, colp, hc)
        if l < num_layers - 1:
            hcur, hc = _upd_call(hcur, aggp, Wl[l], bl[l], Wc[l + 1], bc[l + 1],
                                 n, n_pad, h)
        else:
            hcur = _upd_last_call(hcur, aggp, Wl[l], bl[l], n, n_pad, h)
    return _pool_call(hcur, batch2, Wf1, bf1, Wf2, bf2, n, h)
